# baseline (speedup 1.0000x reference)
import functools

import jax
import jax.numpy as jnp
from jax import lax
from jax.experimental import pallas as pl
from jax.experimental.pallas import tpu as pltpu
from jax.experimental.pallas import tpu_sc as plsc

L = 200
B = 1024
E = 64
LANES = 16
NW = 32
BW = B // NW
CL = 8
NCHUNK = L // CL


@functools.partial(
    pl.kernel,
    out_type=jax.ShapeDtypeStruct((L, B, E), jnp.float32),
    mesh=plsc.VectorSubcoreMesh(core_axis_name="c", subcore_axis_name="s"),
    scratch_types=[
        pltpu.VMEM((CL, BW), jnp.int32),
        pltpu.VMEM((CL, BW, E), jnp.float32),
        pltpu.VMEM((L, E), jnp.float32),
        pltpu.SemaphoreType.DMA,
    ],
    compiler_params=pltpu.CompilerParams(use_tc_tiling_on_sc=False),
)
def _embed_kernel(idx_hbm, table_hbm, pos_hbm, out_hbm, idx_v, rows_v, pos_v, sem):
    wid = lax.axis_index("s") * 2 + lax.axis_index("c")
    b0 = wid * BW

    pltpu.sync_copy(pos_hbm, pos_v)

    def chunk(k, carry):
        l0 = k * CL
        pltpu.sync_copy(idx_hbm.at[pl.ds(l0, CL), pl.ds(b0, BW)], idx_v)
        copies = [
            pltpu.async_copy(table_hbm.at[idx_v.at[l]], rows_v.at[l], sem)
            for l in range(CL)
        ]
        for cp in copies:
            cp.wait()
        for l in range(CL):
            for j in range(E // LANES):
                p = pos_v[l0 + l, pl.ds(j * LANES, LANES)]

                def radd(r, c, l=l, j=j, p=p):
                    sl = pl.ds(j * LANES, LANES)
                    rows_v[l, r, sl] = rows_v[l, r, sl] + p
                    return c

                lax.fori_loop(0, BW, radd, 0, unroll=4)
        pltpu.sync_copy(rows_v, out_hbm.at[pl.ds(l0, CL), pl.ds(b0, BW), :])
        return carry

    lax.fori_loop(0, NCHUNK, chunk, 0)


def kernel(inputs, embedding, posembedding):
    return _embed_kernel(inputs.astype(jnp.int32), embedding, posembedding)


# speedup vs baseline: 1.0603x; 1.0603x over previous
import functools

import jax
import jax.numpy as jnp
from jax import lax
from jax.experimental import pallas as pl
from jax.experimental.pallas import tpu as pltpu
from jax.experimental.pallas import tpu_sc as plsc

L = 200
B = 1024
E = 64
LANES = 16
NBG = 16
BGW = B // NBG
NLG = 2
PH = L // NLG
LP = 5
NU = PH // LP
NQ = E // LANES


@functools.partial(
    pl.kernel,
    out_type=jax.ShapeDtypeStruct((L, B, E), jnp.float32),
    mesh=plsc.VectorSubcoreMesh(core_axis_name="c", subcore_axis_name="s"),
    scratch_types=[
        pltpu.VMEM((L, BGW), jnp.int32),
        pltpu.VMEM((L, E), jnp.float32),
        pltpu.VMEM((LP, BGW, E), jnp.float32),
        pltpu.VMEM((LP, BGW, E), jnp.float32),
        pltpu.SemaphoreType.DMA,
        pltpu.SemaphoreType.DMA,
        pltpu.SemaphoreType.DMA,
        pltpu.SemaphoreType.DMA,
    ],
    compiler_params=pltpu.CompilerParams(use_tc_tiling_on_sc=False),
)
def _embed_kernel(idx_hbm, table_hbm, pos_hbm, out_hbm,
                  idx_v, pos_v, rows0, rows1, gsem0, gsem1, osem0, osem1):
    wid = lax.axis_index("s") * 2 + lax.axis_index("c")
    bg = wid % NBG
    lg = wid // NBG
    b0 = bg * BGW
    l0 = lg * PH

    rows = (rows0, rows1)
    gsem = (gsem0, gsem1)
    osem = (osem0, osem1)

    pltpu.sync_copy(idx_hbm.at[:, pl.ds(b0, BGW)], idx_v)
    pltpu.sync_copy(pos_hbm, pos_v)

    def gathers(u, j):
        lu = l0 + u * LP
        return [
            pltpu.make_async_copy(
                table_hbm.at[idx_v.at[lu + jj]], rows[j].at[jj], gsem[j])
            for jj in range(LP)
        ]

    def out_copy(u, j):
        lu = l0 + u * LP
        return pltpu.make_async_copy(
            rows[j], out_hbm.at[pl.ds(lu, LP), pl.ds(b0, BGW), :], osem[j])

    def unit(u, j):
        @pl.when(u + 1 < NU)
        def _():
            @pl.when(u >= 1)
            def _():
                out_copy(u - 1, 1 - j).wait()
            for cp in gathers(u + 1, 1 - j):
                cp.start()

        for cp in gathers(u, j):
            cp.wait()

        for jj in range(LP):
            lptr = l0 + u * LP + jj
            ps = [pos_v[lptr, pl.ds(q * LANES, LANES)] for q in range(NQ)]

            def radd(r, c, jj=jj, ps=ps):
                for q in range(NQ):
                    sl = pl.ds(q * LANES, LANES)
                    rows[j][jj, r, sl] = rows[j][jj, r, sl] + ps[q]
                return c

            lax.fori_loop(0, BGW, radd, 0, unroll=2)

        out_copy(u, j).start()

    def body(i, carry):
        unit(2 * i, 0)
        unit(2 * i + 1, 1)
        return carry

    for cp in gathers(0, 0):
        cp.start()
    lax.fori_loop(0, NU // 2, body, 0)
    out_copy(NU - 1, 1).wait()


def kernel(inputs, embedding, posembedding):
    return _embed_kernel(inputs.astype(jnp.int32), embedding, posembedding)
